# Initial kernel scaffold; baseline (speedup 1.0000x reference)
import jax
import jax.numpy as jnp
from jax.experimental import pallas as pl


def kernel(x, W_conv, b_conv, W_cls, b_cls, W_bbox, b_bbox):
    raise NotImplementedError("write your pallas kernel here")


# speedup vs baseline: 1.3406x; 1.3406x over previous
import jax
import jax.numpy as jnp
from jax import lax
from jax.experimental import pallas as pl
from jax.experimental.pallas import tpu as pltpu

_C = 96
_H = 128
_W = 128
_HW = _H * _W
_PAD = _W
_NCLS = 15
_NBOX = 60


def _body(x_ref, wc_ref, bc_ref, wcls_ref, bcls_ref, wbb_ref, bbb_ref,
          cls_ref, bbox_ref, sc, sl, sr):
    zpad = jnp.zeros((_C, _PAD), jnp.bfloat16)
    xb = x_ref[0].astype(jnp.bfloat16)

    sc[:, 0:_PAD] = zpad
    sc[:, _PAD + _HW:] = zpad
    sc[:, _PAD:_PAD + _HW] = xb

    col = lax.broadcasted_iota(jnp.int32, (_C, _HW), 1) & (_W - 1)
    xr = sc[:, _PAD + 1:_PAD + 1 + _HW]
    xr = jnp.where(col == _W - 1, jnp.bfloat16(0), xr)
    sr[:, 0:_PAD] = zpad
    sr[:, _PAD + _HW:] = zpad
    sr[:, _PAD:_PAD + _HW] = xr
    xl = sc[:, _PAD - 1:_PAD - 1 + _HW]
    xl = jnp.where(col == 0, jnp.bfloat16(0), xl)
    sl[:, 0:_PAD] = zpad
    sl[:, _PAD + _HW:] = zpad
    sl[:, _PAD:_PAD + _HW] = xl

    srcs = {-1: sl, 0: sc, 1: sr}
    acc = jnp.zeros((_C, _HW), jnp.float32)
    for dy in (-1, 0, 1):
        off = _PAD + dy * _W
        for dx in (-1, 0, 1):
            tap = (dy + 1) * 3 + (dx + 1)
            xt = srcs[dx][:, off:off + _HW]
            acc = acc + lax.dot_general(
                wc_ref[tap], xt, (((1,), (0,)), ((), ())),
                preferred_element_type=jnp.float32)
    h = jnp.maximum(acc + bc_ref[...], 0.0)
    hb = h.astype(jnp.bfloat16)

    cls = lax.dot_general(wcls_ref[...], hb, (((1,), (0,)), ((), ())),
                          preferred_element_type=jnp.float32)
    bbox = lax.dot_general(wbb_ref[...], hb, (((1,), (0,)), ((), ())),
                           preferred_element_type=jnp.float32)
    cls_ref[0] = cls + bcls_ref[...]
    bbox_ref[0] = bbox + bbb_ref[...]


def kernel(x, W_conv, b_conv, W_cls, b_cls, W_bbox, b_bbox):
    n = x.shape[0]
    x2 = x.reshape(n, _C, _HW)
    wc = jnp.transpose(W_conv, (2, 3, 0, 1)).reshape(9, _C, _C)
    wc = wc.astype(jnp.bfloat16)
    wcls = W_cls.reshape(_NCLS, _C).astype(jnp.bfloat16)
    wbb = W_bbox.reshape(_NBOX, _C).astype(jnp.bfloat16)
    bc = b_conv.reshape(_C, 1)
    bcl = b_cls.reshape(_NCLS, 1)
    bbb = b_bbox.reshape(_NBOX, 1)

    logits, bbox = pl.pallas_call(
        _body,
        grid=(n,),
        in_specs=[
            pl.BlockSpec((1, _C, _HW), lambda i: (i, 0, 0)),
            pl.BlockSpec((9, _C, _C), lambda i: (0, 0, 0)),
            pl.BlockSpec((_C, 1), lambda i: (0, 0)),
            pl.BlockSpec((_NCLS, _C), lambda i: (0, 0)),
            pl.BlockSpec((_NCLS, 1), lambda i: (0, 0)),
            pl.BlockSpec((_NBOX, _C), lambda i: (0, 0)),
            pl.BlockSpec((_NBOX, 1), lambda i: (0, 0)),
        ],
        out_specs=[
            pl.BlockSpec((1, _NCLS, _HW), lambda i: (i, 0, 0)),
            pl.BlockSpec((1, _NBOX, _HW), lambda i: (i, 0, 0)),
        ],
        out_shape=[
            jax.ShapeDtypeStruct((n, _NCLS, _HW), jnp.float32),
            jax.ShapeDtypeStruct((n, _NBOX, _HW), jnp.float32),
        ],
        scratch_shapes=[pltpu.VMEM((_C, _HW + 2 * _PAD), jnp.bfloat16)
                        for _ in range(3)],
    )(x2, wc, bc, wcls, bcl, wbb, bbb)

    return (logits.reshape(n, _NCLS, _H, _W),
            bbox.reshape(n, _NBOX, _H, _W))


# speedup vs baseline: 2.3027x; 1.7176x over previous
import jax
import jax.numpy as jnp
from jax import lax
from jax.experimental import pallas as pl
from jax.experimental.pallas import tpu as pltpu

_C = 96
_H = 128
_W = 128
_HW = _H * _W
_PAD = _W
_NCLS = 15
_NBOX = 60


def _body(x_ref, wc_ref, bc_ref, wcls_ref, bcls_ref, wbb_ref, bbb_ref,
          cls_ref, bbox_ref, sc, sl, sr):
    zpad = jnp.zeros((_C, _PAD), jnp.bfloat16)
    xb = x_ref[0].astype(jnp.bfloat16).reshape(_C, _HW)

    sc[:, 0:_PAD] = zpad
    sc[:, _PAD + _HW:] = zpad
    sc[:, _PAD:_PAD + _HW] = xb

    col = lax.broadcasted_iota(jnp.int32, (_C, _HW), 1) & (_W - 1)
    xr = sc[:, _PAD + 1:_PAD + 1 + _HW]
    xr = jnp.where(col == _W - 1, jnp.bfloat16(0), xr)
    sr[:, 0:_PAD] = zpad
    sr[:, _PAD + _HW:] = zpad
    sr[:, _PAD:_PAD + _HW] = xr
    xl = sc[:, _PAD - 1:_PAD - 1 + _HW]
    xl = jnp.where(col == 0, jnp.bfloat16(0), xl)
    sl[:, 0:_PAD] = zpad
    sl[:, _PAD + _HW:] = zpad
    sl[:, _PAD:_PAD + _HW] = xl

    srcs = {-1: sl, 0: sc, 1: sr}
    acc = jnp.zeros((_C, _HW), jnp.float32)
    for dy in (-1, 0, 1):
        off = _PAD + dy * _W
        for dx in (-1, 0, 1):
            tap = (dy + 1) * 3 + (dx + 1)
            xt = srcs[dx][:, off:off + _HW]
            acc = acc + lax.dot_general(
                wc_ref[tap], xt, (((1,), (0,)), ((), ())),
                preferred_element_type=jnp.float32)
    h = jnp.maximum(acc + bc_ref[...], 0.0)
    hb = h.astype(jnp.bfloat16)

    cls = lax.dot_general(wcls_ref[...], hb, (((1,), (0,)), ((), ())),
                          preferred_element_type=jnp.float32)
    bbox = lax.dot_general(wbb_ref[...], hb, (((1,), (0,)), ((), ())),
                           preferred_element_type=jnp.float32)
    cls_ref[0] = (cls + bcls_ref[...]).reshape(_NCLS, _H, _W)
    bbox_ref[0] = (bbox + bbb_ref[...]).reshape(_NBOX, _H, _W)


def kernel(x, W_conv, b_conv, W_cls, b_cls, W_bbox, b_bbox):
    n = x.shape[0]
    wc = jnp.transpose(W_conv, (2, 3, 0, 1)).reshape(9, _C, _C)
    wc = wc.astype(jnp.bfloat16)
    wcls = W_cls.reshape(_NCLS, _C).astype(jnp.bfloat16)
    wbb = W_bbox.reshape(_NBOX, _C).astype(jnp.bfloat16)
    bc = b_conv.reshape(_C, 1)
    bcl = b_cls.reshape(_NCLS, 1)
    bbb = b_bbox.reshape(_NBOX, 1)

    logits, bbox = pl.pallas_call(
        _body,
        grid=(n,),
        in_specs=[
            pl.BlockSpec((1, _C, _H, _W), lambda i: (i, 0, 0, 0)),
            pl.BlockSpec((9, _C, _C), lambda i: (0, 0, 0)),
            pl.BlockSpec((_C, 1), lambda i: (0, 0)),
            pl.BlockSpec((_NCLS, _C), lambda i: (0, 0)),
            pl.BlockSpec((_NCLS, 1), lambda i: (0, 0)),
            pl.BlockSpec((_NBOX, _C), lambda i: (0, 0)),
            pl.BlockSpec((_NBOX, 1), lambda i: (0, 0)),
        ],
        out_specs=[
            pl.BlockSpec((1, _NCLS, _H, _W), lambda i: (i, 0, 0, 0)),
            pl.BlockSpec((1, _NBOX, _H, _W), lambda i: (i, 0, 0, 0)),
        ],
        out_shape=[
            jax.ShapeDtypeStruct((n, _NCLS, _H, _W), jnp.float32),
            jax.ShapeDtypeStruct((n, _NBOX, _H, _W), jnp.float32),
        ],
        scratch_shapes=[pltpu.VMEM((_C, _HW + 2 * _PAD), jnp.bfloat16)
                        for _ in range(3)],
    )(x, wc, bc, wcls, bcl, wbb, bbb)

    return (logits, bbox)


# speedup vs baseline: 2.3483x; 1.0198x over previous
import jax
import jax.numpy as jnp
from jax import lax
from jax.experimental import pallas as pl
from jax.experimental.pallas import tpu as pltpu

_C = 96
_H = 128
_W = 128
_HW = _H * _W
_PAD = _W
_NCLS = 15
_NBOX = 60


def _body(x_ref, wc_ref, bc_ref, wcls_ref, bcls_ref, wbb_ref, bbb_ref,
          cls_ref, bbox_ref, s):
    zpad = jnp.zeros((_C, _PAD), jnp.bfloat16)
    xb = x_ref[0].astype(jnp.bfloat16).reshape(_C, _HW)

    s[0:_C, 0:_PAD] = zpad
    s[0:_C, _PAD + _HW:] = zpad
    s[0:_C, _PAD:_PAD + _HW] = xb

    col = lax.broadcasted_iota(jnp.int32, (_C, _HW), 1) & (_W - 1)
    xl = s[0:_C, _PAD - 1:_PAD - 1 + _HW]
    xl = jnp.where(col == 0, jnp.bfloat16(0), xl)
    s[_C:2 * _C, 0:_PAD] = zpad
    s[_C:2 * _C, _PAD + _HW:] = zpad
    s[_C:2 * _C, _PAD:_PAD + _HW] = xl
    xr = s[0:_C, _PAD + 1:_PAD + 1 + _HW]
    xr = jnp.where(col == _W - 1, jnp.bfloat16(0), xr)
    s[2 * _C:3 * _C, 0:_PAD] = zpad
    s[2 * _C:3 * _C, _PAD + _HW:] = zpad
    s[2 * _C:3 * _C, _PAD:_PAD + _HW] = xr

    acc = None
    for dy in (-1, 0, 1):
        off = _PAD + dy * _W
        part = lax.dot_general(
            wc_ref[dy + 1], s[:, off:off + _HW], (((1,), (0,)), ((), ())),
            preferred_element_type=jnp.float32)
        acc = part if acc is None else acc + part
    h = jnp.maximum(acc + bc_ref[...], 0.0)
    hb = h.astype(jnp.bfloat16)

    cls = lax.dot_general(wcls_ref[...], hb, (((1,), (0,)), ((), ())),
                          preferred_element_type=jnp.float32)
    bbox = lax.dot_general(wbb_ref[...], hb, (((1,), (0,)), ((), ())),
                           preferred_element_type=jnp.float32)
    cls_ref[0] = (cls + bcls_ref[...]).reshape(_NCLS, _H, _W)
    bbox_ref[0] = (bbox + bbb_ref[...]).reshape(_NBOX, _H, _W)


def kernel(x, W_conv, b_conv, W_cls, b_cls, W_bbox, b_bbox):
    n = x.shape[0]
    wt = jnp.transpose(W_conv, (2, 3, 0, 1))
    wc = jnp.concatenate([wt[:, 1], wt[:, 0], wt[:, 2]], axis=-1)
    wc = wc.astype(jnp.bfloat16)
    wcls = W_cls.reshape(_NCLS, _C).astype(jnp.bfloat16)
    wbb = W_bbox.reshape(_NBOX, _C).astype(jnp.bfloat16)
    bc = b_conv.reshape(_C, 1)
    bcl = b_cls.reshape(_NCLS, 1)
    bbb = b_bbox.reshape(_NBOX, 1)

    logits, bbox = pl.pallas_call(
        _body,
        grid=(n,),
        in_specs=[
            pl.BlockSpec((1, _C, _H, _W), lambda i: (i, 0, 0, 0)),
            pl.BlockSpec((3, _C, 3 * _C), lambda i: (0, 0, 0)),
            pl.BlockSpec((_C, 1), lambda i: (0, 0)),
            pl.BlockSpec((_NCLS, _C), lambda i: (0, 0)),
            pl.BlockSpec((_NCLS, 1), lambda i: (0, 0)),
            pl.BlockSpec((_NBOX, _C), lambda i: (0, 0)),
            pl.BlockSpec((_NBOX, 1), lambda i: (0, 0)),
        ],
        out_specs=[
            pl.BlockSpec((1, _NCLS, _H, _W), lambda i: (i, 0, 0, 0)),
            pl.BlockSpec((1, _NBOX, _H, _W), lambda i: (i, 0, 0, 0)),
        ],
        out_shape=[
            jax.ShapeDtypeStruct((n, _NCLS, _H, _W), jnp.float32),
            jax.ShapeDtypeStruct((n, _NBOX, _H, _W), jnp.float32),
        ],
        scratch_shapes=[pltpu.VMEM((3 * _C, _HW + 2 * _PAD), jnp.bfloat16)],
    )(x, wc, bc, wcls, bcl, wbb, bbb)

    return (logits, bbox)
